# baseline (speedup 1.0000x reference)
import functools

import jax
import jax.numpy as jnp
from jax.experimental import pallas as pl

SC = 2
HT = 16
LSLOT = 128


def _body(x_ref, ms_ref, g00_ref, g01_ref, g10_ref, g11_ref, o_ref,
          *, ca, w, ht, cout):
    r = pl.program_id(1)
    r0 = r * ht
    n = ht * w

    slab = x_ref[:, pl.ds(r0, ht + 1), :]
    xt = jnp.transpose(slab, (1, 2, 0))

    def shifted(dy, dx):
        return xt[dy:dy + ht, dx:dx + w, :].reshape(n, ca)

    dot = functools.partial(
        jax.lax.dot_general,
        dimension_numbers=(((1,), (0,)), ((), ())),
        preferred_element_type=jnp.float32)

    p00 = dot(shifted(0, 0), g00_ref[...])
    p01 = dot(shifted(0, 1), g01_ref[...])
    p10 = dot(shifted(1, 0), g10_ref[...])
    p11 = dot(shifted(1, 1), g11_ref[...])

    y_ee = p00[:, 0:256]
    y_eo = p00[:, 256:512] + p01[:, 0:256]
    y_oe = p00[:, 512:768] + p10[:, 0:256]
    y_oo = p00[:, 768:1024] + p01[:, 256:512] + p10[:, 256:512] + p11

    def pick(y, dy, dx):
        m = ms_ref[0, dy, dx][:, :, None]
        d = y[:, 0:cout].reshape(ht, w, cout)
        zl = y[:, LSLOT:LSLOT + cout].reshape(ht, w, cout)
        return zl + d * m

    o_ref[0::2, 0::2, :] = pick(y_ee, 0, 0)
    o_ref[0::2, 1::2, :] = pick(y_eo, 0, 1)
    o_ref[1::2, 0::2, :] = pick(y_oe, 1, 0)
    o_ref[1::2, 1::2, :] = pick(y_oo, 1, 1)


def kernel(inx, mask, inv_mask, high_w, high_b, low1_w, low1_b, low2_w, low2_b):
    del inv_mask
    b, cin, h, w = inx.shape
    cout = high_w.shape[1]
    ca = cin + 1
    ht = HT
    nr = h // ht

    l1 = low1_w[:, :, 0, 0]
    w_eff = jnp.einsum('mc,mnyx->cnyx', l1, low2_w)
    b_eff = jnp.einsum('m,mnyx->nyx', low1_b, low2_w)

    def tap(ky, kx, add_flat):
        hi_b = high_b if add_flat else jnp.zeros_like(high_b)
        lo_b = b_eff[:, ky, kx] + (low2_b if add_flat else jnp.zeros_like(low2_b))
        hi = jnp.concatenate([high_w[:, :, ky, kx], hi_b[None, :]], axis=0)
        lo = jnp.concatenate([w_eff[:, :, ky, kx], lo_b[None, :]], axis=0)
        pad = jnp.zeros((ca, LSLOT - cout), dtype=inx.dtype)
        return jnp.concatenate([hi - lo, pad, lo, pad], axis=1)

    g00 = jnp.concatenate(
        [tap(1, 1, True), tap(1, 2, True), tap(2, 1, True), tap(2, 2, True)],
        axis=1)
    g01 = jnp.concatenate([tap(1, 0, False), tap(2, 0, False)], axis=1)
    g10 = jnp.concatenate([tap(0, 1, False), tap(0, 2, False)], axis=1)
    g11 = tap(0, 0, False)

    x_aug = jnp.concatenate([inx, jnp.ones((b, 1, h, w), inx.dtype)], axis=1)
    x_pad = jnp.pad(x_aug, ((0, 0), (0, 0), (0, 1), (0, 1)))
    x_pad = x_pad.astype(jnp.bfloat16)
    g00, g01, g10, g11 = (g.astype(jnp.bfloat16) for g in (g00, g01, g10, g11))

    m6 = mask.reshape(b, h, SC, w, SC)
    ms = jnp.transpose(m6, (0, 2, 4, 1, 3))

    grid = (b, nr)
    y = pl.pallas_call(
        functools.partial(_body, ca=ca, w=w, ht=ht, cout=cout),
        grid=grid,
        in_specs=[
            pl.BlockSpec((None, ca, h + 1, w + 1), lambda bi, r: (bi, 0, 0, 0)),
            pl.BlockSpec((1, SC, SC, ht, w), lambda bi, r: (bi, 0, 0, r, 0)),
            pl.BlockSpec((ca, 4 * 256), lambda bi, r: (0, 0)),
            pl.BlockSpec((ca, 2 * 256), lambda bi, r: (0, 0)),
            pl.BlockSpec((ca, 2 * 256), lambda bi, r: (0, 0)),
            pl.BlockSpec((ca, 256), lambda bi, r: (0, 0)),
        ],
        out_specs=pl.BlockSpec((None, SC * ht, SC * w, cout),
                               lambda bi, r: (bi, r, 0, 0)),
        out_shape=jax.ShapeDtypeStruct((b, SC * h, SC * w, cout), jnp.float32),
    )(x_pad, ms, g00, g01, g10, g11)
    return jnp.transpose(y, (0, 3, 1, 2))


# speedup vs baseline: 1.0426x; 1.0426x over previous
import functools

import jax
import jax.numpy as jnp
from jax.experimental import pallas as pl

SC = 2
HT = 16
LSLOT = 128


def _body(x_ref, ms_ref, g00_ref, g01_ref, g10_ref, g11_ref, o_ref,
          *, ca, w, ht, cout):
    r = pl.program_id(1)
    r0 = r * ht
    n = ht * w

    def shifted(dy, dx):
        blk = x_ref[pl.ds(r0 + dy, ht), dx:dx + w, :]
        return blk.reshape(n, ca)

    dot = functools.partial(
        jax.lax.dot_general,
        dimension_numbers=(((1,), (0,)), ((), ())),
        preferred_element_type=jnp.float32)

    p00 = dot(shifted(0, 0), g00_ref[...])
    p01 = dot(shifted(0, 1), g01_ref[...])
    p10 = dot(shifted(1, 0), g10_ref[...])
    p11 = dot(shifted(1, 1), g11_ref[...])

    y_ee = p00[:, 0:256]
    y_eo = p00[:, 256:512] + p01[:, 0:256]
    y_oe = p00[:, 512:768] + p10[:, 0:256]
    y_oo = p00[:, 768:1024] + p01[:, 256:512] + p10[:, 256:512] + p11

    def pick(y, dy, dx):
        m = ms_ref[0, dy, dx][:, :, None]
        d = y[:, 0:cout].reshape(ht, w, cout)
        zl = y[:, LSLOT:LSLOT + cout].reshape(ht, w, cout)
        return zl + d * m

    o_ref[0::2, 0::2, :] = pick(y_ee, 0, 0)
    o_ref[0::2, 1::2, :] = pick(y_eo, 0, 1)
    o_ref[1::2, 0::2, :] = pick(y_oe, 1, 0)
    o_ref[1::2, 1::2, :] = pick(y_oo, 1, 1)


def kernel(inx, mask, inv_mask, high_w, high_b, low1_w, low1_b, low2_w, low2_b):
    del inv_mask
    b, cin, h, w = inx.shape
    cout = high_w.shape[1]
    ca = cin + 1
    ht = HT
    nr = h // ht

    l1 = low1_w[:, :, 0, 0]
    w_eff = jnp.einsum('mc,mnyx->cnyx', l1, low2_w)
    b_eff = jnp.einsum('m,mnyx->nyx', low1_b, low2_w)

    def tap(ky, kx, add_flat):
        hi_b = high_b if add_flat else jnp.zeros_like(high_b)
        lo_b = b_eff[:, ky, kx] + (low2_b if add_flat else jnp.zeros_like(low2_b))
        hi = jnp.concatenate([high_w[:, :, ky, kx], hi_b[None, :]], axis=0)
        lo = jnp.concatenate([w_eff[:, :, ky, kx], lo_b[None, :]], axis=0)
        pad = jnp.zeros((ca, LSLOT - cout), dtype=inx.dtype)
        return jnp.concatenate([hi - lo, pad, lo, pad], axis=1)

    g00 = jnp.concatenate(
        [tap(1, 1, True), tap(1, 2, True), tap(2, 1, True), tap(2, 2, True)],
        axis=1)
    g01 = jnp.concatenate([tap(1, 0, False), tap(2, 0, False)], axis=1)
    g10 = jnp.concatenate([tap(0, 1, False), tap(0, 2, False)], axis=1)
    g11 = tap(0, 0, False)

    x_aug = jnp.concatenate([inx, jnp.ones((b, 1, h, w), inx.dtype)], axis=1)
    x_cm = jnp.pad(x_aug, ((0, 0), (0, 0), (0, 1), (0, 1))).astype(jnp.bfloat16)
    x_pad = jnp.transpose(x_cm, (0, 2, 3, 1))
    g00, g01, g10, g11 = (g.astype(jnp.bfloat16) for g in (g00, g01, g10, g11))

    m6 = mask.reshape(b, h, SC, w, SC)
    ms = jnp.transpose(m6, (0, 2, 4, 1, 3))

    grid = (b, nr)
    y = pl.pallas_call(
        functools.partial(_body, ca=ca, w=w, ht=ht, cout=cout),
        grid=grid,
        in_specs=[
            pl.BlockSpec((None, h + 1, w + 1, ca), lambda bi, r: (bi, 0, 0, 0)),
            pl.BlockSpec((1, SC, SC, ht, w), lambda bi, r: (bi, 0, 0, r, 0)),
            pl.BlockSpec((ca, 4 * 256), lambda bi, r: (0, 0)),
            pl.BlockSpec((ca, 2 * 256), lambda bi, r: (0, 0)),
            pl.BlockSpec((ca, 2 * 256), lambda bi, r: (0, 0)),
            pl.BlockSpec((ca, 256), lambda bi, r: (0, 0)),
        ],
        out_specs=pl.BlockSpec((None, SC * ht, SC * w, cout),
                               lambda bi, r: (bi, r, 0, 0)),
        out_shape=jax.ShapeDtypeStruct((b, SC * h, SC * w, cout), jnp.float32),
    )(x_pad, ms, g00, g01, g10, g11)
    return jnp.transpose(y, (0, 3, 1, 2))


# speedup vs baseline: 1.0529x; 1.0098x over previous
import functools

import jax
import jax.numpy as jnp
from jax.experimental import pallas as pl

SC = 2
HT = 16
LSLOT = 128


def _body(x_ref, ms_ref, g00_ref, g01_ref, g10_ref, g11_ref, o_ref,
          *, ca, w, ht, cout):
    r = pl.program_id(1)
    r0 = r * ht
    n = ht * w

    def shifted(dy, dx):
        blk = x_ref[pl.ds(r0 + dy, ht), dx:dx + w, :]
        return blk.reshape(n, ca)

    dot = functools.partial(
        jax.lax.dot_general,
        dimension_numbers=(((1,), (0,)), ((), ())),
        preferred_element_type=jnp.float32)

    p00 = dot(shifted(0, 0), g00_ref[...])
    p01 = dot(shifted(0, 1), g01_ref[...])
    p10 = dot(shifted(1, 0), g10_ref[...])
    p11 = dot(shifted(1, 1), g11_ref[...])

    y_ee = p00[:, 0:256]
    y_eo = p00[:, 256:512] + p01[:, 0:256]
    y_oe = p00[:, 512:768] + p10[:, 0:256]
    y_oo = p00[:, 768:1024] + p01[:, 256:512] + p10[:, 256:512] + p11

    def pick(y, dy, dx):
        m = ms_ref[0, dy, dx][:, :, None]
        d = y[:, 0:cout].reshape(ht, w, cout)
        zl = y[:, LSLOT:LSLOT + cout].reshape(ht, w, cout)
        return zl + d * m

    o_ref[0::2, 0::2, :] = pick(y_ee, 0, 0)
    o_ref[0::2, 1::2, :] = pick(y_eo, 0, 1)
    o_ref[1::2, 0::2, :] = pick(y_oe, 1, 0)
    o_ref[1::2, 1::2, :] = pick(y_oo, 1, 1)


def kernel(inx, mask, inv_mask, high_w, high_b, low1_w, low1_b, low2_w, low2_b):
    del inv_mask
    b, cin, h, w = inx.shape
    cout = high_w.shape[1]
    ca = cin + 1
    ht = HT
    nr = h // ht

    l1 = low1_w[:, :, 0, 0]
    w_eff = jnp.einsum('mc,mnyx->cnyx', l1, low2_w)
    b_eff = jnp.einsum('m,mnyx->nyx', low1_b, low2_w)
    fl = jnp.array([[0, 0, 0], [0, 1, 1], [0, 1, 1]], inx.dtype)
    low_row = b_eff + low2_b[:, None, None] * fl
    high_row = high_b[:, None, None] * fl
    top = jnp.concatenate([high_w - w_eff, w_eff], axis=1)
    row = jnp.concatenate([high_row - low_row, low_row])[None]
    t9 = jnp.concatenate([top, row], axis=0)
    t9 = t9.reshape(ca, 2, cout, 3, 3)
    t9 = jnp.pad(t9, ((0, 0), (0, 0), (0, LSLOT - cout), (0, 0), (0, 0)))
    t9 = t9.reshape(ca, 2 * LSLOT, 3, 3).astype(jnp.bfloat16)
    g00 = jnp.concatenate(
        [t9[..., 1, 1], t9[..., 1, 2], t9[..., 2, 1], t9[..., 2, 2]], axis=1)
    g01 = jnp.concatenate([t9[..., 1, 0], t9[..., 2, 0]], axis=1)
    g10 = jnp.concatenate([t9[..., 0, 1], t9[..., 0, 2]], axis=1)
    g11 = t9[..., 0, 0]

    x_aug = jnp.concatenate([inx, jnp.ones((b, 1, h, w), inx.dtype)], axis=1)
    x_cm = jnp.pad(x_aug, ((0, 0), (0, 0), (0, 1), (0, 1))).astype(jnp.bfloat16)
    x_pad = jnp.transpose(x_cm, (0, 2, 3, 1))

    m6 = mask.astype(jnp.bfloat16).reshape(b, h, SC, w, SC)
    ms = jnp.transpose(m6, (0, 2, 4, 1, 3))

    grid = (b, nr)
    y = pl.pallas_call(
        functools.partial(_body, ca=ca, w=w, ht=ht, cout=cout),
        grid=grid,
        in_specs=[
            pl.BlockSpec((None, h + 1, w + 1, ca), lambda bi, r: (bi, 0, 0, 0)),
            pl.BlockSpec((1, SC, SC, ht, w), lambda bi, r: (bi, 0, 0, r, 0)),
            pl.BlockSpec((ca, 4 * 256), lambda bi, r: (0, 0)),
            pl.BlockSpec((ca, 2 * 256), lambda bi, r: (0, 0)),
            pl.BlockSpec((ca, 2 * 256), lambda bi, r: (0, 0)),
            pl.BlockSpec((ca, 256), lambda bi, r: (0, 0)),
        ],
        out_specs=pl.BlockSpec((None, SC * ht, SC * w, cout),
                               lambda bi, r: (bi, r, 0, 0)),
        out_shape=jax.ShapeDtypeStruct((b, SC * h, SC * w, cout), jnp.float32),
    )(x_pad, ms, g00, g01, g10, g11)
    return jnp.transpose(y, (0, 3, 1, 2))
